# baseline (speedup 1.0000x reference)
import functools

import jax
import jax.numpy as jnp
from jax import lax
from jax.experimental import pallas as pl
from jax.experimental.pallas import tpu as pltpu
from jax.experimental.pallas import tpu_sc as plsc

_NC = 2
_NS = 16
_NW = _NC * _NS

_BL = 128
_D = 64
_TB = 8 * 8 * _BL


def _body(table_hbm, idxt_hbm, out_hbm, idx_v, g0, g1, t0, t1,
          gsem, osem0, osem1):
    seq, batch = idxt_hbm.shape
    wid = lax.axis_index("s") * _NC + lax.axis_index("c")
    nbb = batch // _BL // _NW
    gbufs = (g0, g1)
    tbufs = (t0, t1)
    osems = (osem0, osem1)
    iota = lax.iota(jnp.int32, 16)
    patt = (iota // 8) * 1024 + (iota % 8) * 128

    def transpose_block(gb, tb):
        for b in range(_BL):
            for k in range(_D // 16):
                val = gb[b, pl.ds(16 * k, 16)]
                plsc.store_scatter(tb, [patt + (2048 * k + b)], val)

    @pl.loop(0, nbb)
    def _bblock(bbi):
        bbg = wid * nbb + bbi
        pltpu.sync_copy(idxt_hbm.at[:, pl.ds(bbg * _BL, _BL)], idx_v)

        def gather(s, gb):
            return pltpu.async_copy(table_hbm.at[idx_v.at[s]], gb, gsem)

        def out_off(s, c8):
            return s * (_D * batch) + c8 * (8 * batch) + bbg * (8 * _BL)

        gather(0, gbufs[0])
        gather(1, gbufs[1])

        @pl.loop(0, seq, step=2)
        def _pair(p):
            for sb in range(2):
                s = p + sb
                gb, tb, osem = gbufs[sb], tbufs[sb], osems[sb]

                @pl.when(s >= 2)
                def _():
                    for c8 in range(8):
                        pltpu.make_async_copy(
                            tb.at[pl.ds(c8 * 1024, 1024)],
                            out_hbm.at[pl.ds(out_off(s - 2, c8), 1024)],
                            osem,
                        ).wait()

                pltpu.make_async_copy(
                    table_hbm.at[idx_v.at[s]], gb, gsem
                ).wait()
                transpose_block(gb, tb)
                for c8 in range(8):
                    pltpu.async_copy(
                        tb.at[pl.ds(c8 * 1024, 1024)],
                        out_hbm.at[pl.ds(out_off(s, c8), 1024)],
                        osem,
                    )

                @pl.when(s + 2 < seq)
                def _():
                    gather(s + 2, gb)

        for sb in range(2):
            for c8 in range(8):
                pltpu.make_async_copy(
                    tbufs[sb].at[pl.ds(c8 * 1024, 1024)],
                    out_hbm.at[pl.ds(out_off(seq - 2 + sb, c8), 1024)],
                    osems[sb],
                ).wait()


@jax.jit
def _gather(averages, idxt):
    seq, batch = idxt.shape
    d = averages.shape[1]
    mesh = plsc.VectorSubcoreMesh(core_axis_name="c", subcore_axis_name="s")
    return pl.kernel(
        _body,
        out_type=jax.ShapeDtypeStruct((seq * d * batch,), averages.dtype),
        mesh=mesh,
        scratch_types=[
            pltpu.VMEM((seq, _BL), jnp.int32),
            pltpu.VMEM((_BL, d), jnp.float32),
            pltpu.VMEM((_BL, d), jnp.float32),
            pltpu.VMEM((_TB,), jnp.float32),
            pltpu.VMEM((_TB,), jnp.float32),
            pltpu.SemaphoreType.DMA,
            pltpu.SemaphoreType.DMA,
            pltpu.SemaphoreType.DMA,
        ],
        compiler_params=pltpu.CompilerParams(
            use_tc_tiling_on_sc=False,
            needs_layout_passes=False,
            disable_bounds_checks=True,
        ),
    )(averages, idxt)


def kernel(indicies, averages):
    batch, seq = indicies.shape
    d = averages.shape[1]
    av_flat = jax.lax.optimization_barrier(averages.reshape(-1))
    av = av_flat.reshape(averages.shape)
    idxt = indicies.astype(jnp.int32).T
    flat = _gather(av, idxt)
    out5 = flat.reshape(seq, d // 8, batch // _BL, 8, _BL)
    y = out5.transpose(2, 4, 0, 1, 3).reshape(batch, seq, d)
    return y


# speedup vs baseline: 2.4707x; 2.4707x over previous
import functools

import jax
import jax.numpy as jnp
from jax import lax
from jax.experimental import pallas as pl
from jax.experimental.pallas import tpu as pltpu
from jax.experimental.pallas import tpu_sc as plsc

_NC = 2
_NS = 16
_NW = _NC * _NS

_GB = 4


def _body(table_hbm, idx_hbm, out_hbm, idx_v, rows_v, gsem, osem0, osem1):
    seq = idx_hbm.shape[1]
    wid = lax.axis_index("s") * _NC + lax.axis_index("c")
    nrows = idx_hbm.shape[0] // _NW
    b0 = wid * nrows
    ng = nrows // _GB
    osems = (osem0, osem1)
    splits = [(0, 128), (128, seq - 128)] if seq > 128 else [(0, seq)]

    def fire_gathers(b, row0):
        return [
            pltpu.async_copy(
                table_hbm.at[idx_v.at[b, r, pl.ds(lo, ln)]],
                rows_v.at[b, r, pl.ds(lo, ln)],
                gsem,
            )
            for r in range(_GB)
            for (lo, ln) in splits
        ]

    def drain_gathers(b):
        for r in range(_GB):
            for (lo, ln) in splits:
                pltpu.make_async_copy(
                    table_hbm.at[idx_v.at[b, r, pl.ds(lo, ln)]],
                    rows_v.at[b, r, pl.ds(lo, ln)],
                    gsem,
                ).wait()

    def out_slice(row0):
        return out_hbm.at[pl.ds(row0, _GB), :, pl.ds(0, 64)]

    pltpu.sync_copy(idx_hbm.at[pl.ds(b0, _GB)], idx_v.at[0])
    fire_gathers(0, b0)
    pltpu.sync_copy(idx_hbm.at[pl.ds(b0 + _GB, _GB)], idx_v.at[1])

    @pl.loop(0, ng, step=2)
    def _pair(p):
        for b in range(2):
            g = p + b
            row0 = b0 + g * _GB

            @pl.when(g >= 1)
            def _():
                pltpu.make_async_copy(
                    rows_v.at[1 - b], out_slice(row0 - _GB), osems[1 - b]
                ).wait()

            @pl.when(g + 1 < ng)
            def _():
                fire_gathers(1 - b, row0 + _GB)

            drain_gathers(b)
            pltpu.async_copy(rows_v.at[b], out_slice(row0), osems[b])

            @pl.when(g + 2 < ng)
            def _():
                pltpu.sync_copy(
                    idx_hbm.at[pl.ds(row0 + 2 * _GB, _GB)], idx_v.at[b]
                )

    bl = (ng - 1) % 2
    pltpu.make_async_copy(
        rows_v.at[bl], out_hbm.at[pl.ds(b0, _GB), :, pl.ds(0, 64)], osems[bl]
    ).wait()


@jax.jit
def _gather(averages, idx2d):
    batch, seq = idx2d.shape
    d = averages.shape[1]
    mesh = plsc.VectorSubcoreMesh(core_axis_name="c", subcore_axis_name="s")
    return pl.kernel(
        _body,
        out_type=jax.ShapeDtypeStruct((batch, seq, 128), averages.dtype),
        mesh=mesh,
        scratch_types=[
            pltpu.VMEM((2, _GB, seq), jnp.int32),
            pltpu.VMEM((2, _GB, seq, d), jnp.float32),
            pltpu.SemaphoreType.DMA,
            pltpu.SemaphoreType.DMA,
            pltpu.SemaphoreType.DMA,
        ],
        compiler_params=pltpu.CompilerParams(use_tc_tiling_on_sc=False),
    )(averages, idx2d)


def kernel(indicies, averages):
    av_flat = jax.lax.optimization_barrier(averages.reshape(-1))
    idx_flat = jax.lax.optimization_barrier(
        indicies.astype(jnp.int32).reshape(-1)
    )
    av = av_flat.reshape(averages.shape)
    idx = idx_flat.reshape(indicies.shape)
    return _gather(av, idx)[..., :64]
